# baseline (speedup 1.0000x reference)
import functools

import jax
import jax.numpy as jnp
from jax import lax
from jax.experimental import pallas as pl


def _fused_body(nexp, sblk, x_ref, gw_ref, gb_ref, ew_ref, eb_ref,
                topw_ref, out_ref):
    xx = x_ref[0]
    bt = xx.shape[0]
    logits = jnp.dot(xx, gw_ref[...], preferred_element_type=jnp.float32)
    logits = logits + gb_ref[...]
    lanes = lax.broadcasted_iota(jnp.int32, logits.shape, 1)
    logits = jnp.where(lanes < nexp, logits, -jnp.inf)
    m = jnp.max(logits, axis=1, keepdims=True)
    ex = jnp.exp(logits - m)
    s = jnp.sum(ex, axis=1, keepdims=True)
    w = ex / s
    m1 = jnp.max(w, axis=1, keepdims=True)
    i1 = jnp.min(jnp.where(w == m1, lanes, 128), axis=1, keepdims=True)
    w2 = jnp.where(lanes == i1, -1.0, w)
    m2 = jnp.max(w2, axis=1, keepdims=True)
    i2 = jnp.min(jnp.where(w2 == m2, lanes, 128), axis=1, keepdims=True)
    topw_ref[0] = jnp.concatenate([m1, m2], axis=1)
    xb = xx.astype(jnp.bfloat16)
    acc1 = jnp.zeros((bt, out_ref.shape[3]), jnp.bfloat16)
    acc2 = jnp.zeros((bt, out_ref.shape[3]), jnp.bfloat16)
    for e in range(nexp):
        oe = jnp.dot(xb, ew_ref[e],
                     preferred_element_type=jnp.float32).astype(jnp.bfloat16)
        oe = oe + eb_ref[e][None, :]
        acc1 = jnp.where(i1 == e, oe, acc1)
        acc2 = jnp.where(i2 == e, oe, acc2)
    out_ref[0] = jnp.stack([acc1, acc2], axis=1).astype(jnp.float32)


def kernel(x, gate_w, gate_b, expert_w, expert_b):
    B, S, EMB = x.shape
    NE, _, HID = expert_w.shape
    BT = 512
    gw = jnp.pad(gate_w, ((0, 0), (0, 128 - NE)))
    gb = jnp.pad(gate_b, (0, 128 - NE)).reshape(1, 128)
    ew16 = expert_w.astype(jnp.bfloat16)
    eb16 = expert_b.astype(jnp.bfloat16)

    topw, out = pl.pallas_call(
        functools.partial(_fused_body, NE, S // BT),
        grid=(B, S // BT),
        in_specs=[
            pl.BlockSpec((1, BT, EMB), lambda b, t: (b, t, 0)),
            pl.BlockSpec((EMB, 128), lambda b, t: (0, 0)),
            pl.BlockSpec((1, 128), lambda b, t: (0, 0)),
            pl.BlockSpec((NE, EMB, HID), lambda b, t: (0, 0, 0)),
            pl.BlockSpec((NE, HID), lambda b, t: (0, 0)),
        ],
        out_specs=[
            pl.BlockSpec((1, BT, 2), lambda b, t: (b, t, 0)),
            pl.BlockSpec((1, BT, 2, HID), lambda b, t: (b, t, 0, 0)),
        ],
        out_shape=[
            jax.ShapeDtypeStruct((B, S, 2), jnp.float32),
            jax.ShapeDtypeStruct((B, S, 2, HID), jnp.float32),
        ],
    )(x, gw, gb, ew16, eb16)

    return topw, out
